# Initial kernel scaffold; baseline (speedup 1.0000x reference)
import functools

import jax
import jax.numpy as jnp
from jax import lax
from jax.experimental import pallas as pl
from jax.experimental.pallas import tpu as pltpu
from jax.experimental.pallas import tpu_sc as plsc

D = 128
WIDE = 144
NW = 32
CH = 128
LANES = 16


def _sc_segsum(nchunk, npad, x, src3, dst3, w3, zrows):
    mesh = plsc.VectorSubcoreMesh(core_axis_name="c", subcore_axis_name="s")
    rows_per_tile = npad // 16

    @functools.partial(
        pl.kernel,
        out_type=jax.ShapeDtypeStruct((2, npad, WIDE), jnp.float32),
        mesh=mesh,
        scratch_types=[
            pltpu.VMEM((nchunk, CH), jnp.int32),
            pltpu.VMEM((nchunk, CH), jnp.int32),
            pltpu.VMEM((nchunk, CH), jnp.float32),
            pltpu.VMEM((CH, D), jnp.float32),
            pltpu.VMEM((CH, WIDE), jnp.float32),
            pltpu.VMEM_SHARED((npad, WIDE), jnp.float32),
            pltpu.SemaphoreType.DMA,
        ],
    )
    def k(x_hbm, src_hbm, dst_hbm, w_hbm, z_hbm, out_hbm,
          src_v, dst_v, w_v, rows_v, sc_v, acc, sem):
        c = lax.axis_index("c")
        s = lax.axis_index("s")
        wid = c * 16 + s

        base = s * rows_per_tile
        for i in range(rows_per_tile // CH):
            pltpu.sync_copy(z_hbm, acc.at[pl.ds(base + i * CH, CH)])

        pltpu.sync_copy(src_hbm.at[wid], src_v)
        pltpu.sync_copy(dst_hbm.at[wid], dst_v)
        pltpu.sync_copy(w_hbm.at[wid], w_v)
        plsc.subcore_barrier()

        @pl.loop(0, nchunk)
        def _(j):
            pltpu.async_copy(x_hbm.at[src_v.at[j]], rows_v, sem).wait()

            jj = jnp.full((LANES,), j, jnp.int32)

            @pl.loop(0, CH)
            def _(e):
                wv = plsc.load_gather(w_v, [jj, jnp.full((LANES,), e, jnp.int32)])
                for t in range(D // LANES):
                    sl = pl.ds(t * LANES, LANES)
                    sc_v[e, sl] = rows_v[e, sl] * wv
                sc_v[e, pl.ds(D, LANES)] = wv

            pltpu.sync_copy(sc_v, acc.at[dst_v.at[j]], add=True)

        plsc.subcore_barrier()

        for i in range(rows_per_tile // CH):
            sl = pl.ds(base + i * CH, CH)
            pltpu.sync_copy(acc.at[sl], out_hbm.at[c].at[sl])

    return k(x, src3, dst3, w3, zrows)


def _combine_body(x_ref, p_ref, ws_ref, wn_ref, bs_ref, bn_ref, o_ref):
    p0 = p_ref[0]
    p1 = p_ref[1]
    neigh = p0[:, :D] + p1[:, :D]
    wsum = p0[:, D:D + 1] + p1[:, D:D + 1]
    h = neigh / jnp.maximum(wsum, 1.0)
    o_ref[...] = (
        jnp.dot(x_ref[...], ws_ref[...], preferred_element_type=jnp.float32)
        + jnp.dot(h, wn_ref[...], preferred_element_type=jnp.float32)
        + bs_ref[...] + bn_ref[...]
    ) * 0.5


def _combine(npad, xp, partials, WsT, WnT, bs, bn):
    blk = 1024
    return pl.pallas_call(
        _combine_body,
        grid=(npad // blk,),
        in_specs=[
            pl.BlockSpec((blk, D), lambda i: (i, 0)),
            pl.BlockSpec((2, blk, WIDE), lambda i: (0, i, 0)),
            pl.BlockSpec((D, D), lambda i: (0, 0)),
            pl.BlockSpec((D, D), lambda i: (0, 0)),
            pl.BlockSpec((1, D), lambda i: (0, 0)),
            pl.BlockSpec((1, D), lambda i: (0, 0)),
        ],
        out_specs=pl.BlockSpec((blk, D), lambda i: (i, 0)),
        out_shape=jax.ShapeDtypeStruct((npad, D), jnp.float32),
    )(xp, partials, WsT, WnT, bs, bn)


def kernel(x, edge_index, edge_weight, W_self, b_self, W_neigh, b_neigh):
    n = x.shape[0]
    e = edge_weight.shape[0]

    per_tile = -(-e // (NW * CH)) * CH
    nchunk = per_tile // CH
    epad = NW * per_tile
    npad = -(-n // (16 * CH)) * (16 * CH)

    src3 = jnp.pad(edge_index[0], (0, epad - e)).reshape(NW, nchunk, CH)
    dst3 = jnp.pad(edge_index[1], (0, epad - e)).reshape(NW, nchunk, CH)
    w3 = jnp.pad(edge_weight, (0, epad - e)).reshape(NW, nchunk, CH)
    zrows = jnp.zeros((CH, WIDE), jnp.float32)

    partials = _sc_segsum(nchunk, npad, x, src3, dst3, w3, zrows)

    xp = jnp.pad(x, ((0, npad - n), (0, 0)))
    out = _combine(npad, xp, partials, W_self.T, W_neigh.T,
                   b_self[None, :], b_neigh[None, :])
    return out[:n]


# speedup vs baseline: 8.6222x; 8.6222x over previous
import dataclasses
import functools

import jax
import jax.numpy as jnp
from jax import lax
from jax.experimental import pallas as pl
from jax.experimental.pallas import tpu as pltpu
from jax.experimental.pallas import tpu_sc as plsc

D = 128
NW = 32
CH = 128
BLKCH = 8
LANES = 16

_BCAST_DNUMS = lax.GatherDimensionNumbers(
    offset_dims=(), collapsed_slice_dims=(0,), start_index_map=(0,))


def _bcast_lane(v16, k):
    idx = jnp.full((LANES, 1), k, jnp.int32)
    return lax.gather(v16, idx, _BCAST_DNUMS, slice_sizes=(1,),
                      mode=lax.GatherScatterMode.PROMISE_IN_BOUNDS)


def _sc_segsum(nchunk, npad, x, src3, dst3, w3, zrows):
    mesh = plsc.VectorSubcoreMesh(core_axis_name="c", subcore_axis_name="s")
    rows_per_tile = npad // 16
    cp = pltpu.CompilerParams()
    if "needs_layout_passes" in pltpu.CompilerParams.__dataclass_fields__:
        cp = dataclasses.replace(cp, needs_layout_passes=False)

    @functools.partial(
        pl.kernel,
        out_type=(
            jax.ShapeDtypeStruct((2, npad, D), jnp.float32),
            jax.ShapeDtypeStruct((NW, npad), jnp.float32),
        ),
        mesh=mesh,
        scratch_types=[
            pltpu.VMEM((BLKCH, CH), jnp.int32),
            pltpu.VMEM((BLKCH, CH), jnp.int32),
            pltpu.VMEM((BLKCH, CH), jnp.float32),
            pltpu.VMEM((CH, D), jnp.float32),
            pltpu.VMEM((CH, D), jnp.float32),
            pltpu.VMEM((npad,), jnp.float32),
            pltpu.VMEM_SHARED((npad, D), jnp.float32),
            pltpu.SemaphoreType.DMA,
            pltpu.SemaphoreType.DMA,
            pltpu.SemaphoreType.DMA,
            pltpu.SemaphoreType.DMA,
        ],
        compiler_params=cp,
    )
    def k(x_hbm, src_hbm, dst_hbm, w_hbm, z_hbm, out_hbm, ws_hbm,
          src_v, dst_v, w_v, rows0_v, rows1_v, ws_v, acc,
          sem_g0, sem_g1, sem_s0, sem_s1):
        c = lax.axis_index("c")
        s = lax.axis_index("s")
        wid = c * 16 + s

        base = s * rows_per_tile
        for i in range(rows_per_tile // CH):
            pltpu.sync_copy(z_hbm, acc.at[pl.ds(base + i * CH, CH)])

        zv = jnp.zeros((LANES,), jnp.float32)

        @pl.loop(0, npad, step=LANES)
        def _(i):
            ws_v[pl.ds(i, LANES)] = zv

        plsc.subcore_barrier()

        lane0 = lax.iota(jnp.int32, LANES) == 0

        bufs = (rows0_v, rows1_v)
        gsems = (sem_g0, sem_g1)
        ssems = (sem_s0, sem_s1)

        def _scale_chunk(buf, r):
            @pl.loop(0, CH, step=LANES)
            def _(g):
                w16 = w_v[r, pl.ds(g, LANES)]
                d16 = dst_v[r, pl.ds(g, LANES)]
                for k in range(LANES):
                    wv = _bcast_lane(w16, k)
                    dv = _bcast_lane(d16, k)
                    e = g + k
                    for t in range(D // LANES):
                        sl = pl.ds(t * LANES, LANES)
                        buf[e, sl] = buf[e, sl] * wv
                    plsc.addupdate_scatter(ws_v, [dv], wv, mask=lane0)

        @pl.loop(0, nchunk // BLKCH)
        def _(b):
            @pl.when(b > 0)
            def _():
                for r in range(BLKCH - 2, BLKCH):
                    pltpu.make_async_copy(
                        bufs[r % 2], acc.at[dst_v.at[r]], ssems[r % 2]).wait()

            blk = pl.ds(b * BLKCH, BLKCH)
            pltpu.sync_copy(src_hbm.at[wid].at[blk], src_v)
            pltpu.sync_copy(dst_hbm.at[wid].at[blk], dst_v)
            pltpu.sync_copy(w_hbm.at[wid].at[blk], w_v)

            for r in range(0, BLKCH, 2):
                if r >= 2:
                    pltpu.make_async_copy(
                        bufs[0], acc.at[dst_v.at[r - 2]], sem_s0).wait()
                pltpu.async_copy(x_hbm.at[src_v.at[r]], bufs[0], sem_g0)
                if r >= 2:
                    pltpu.make_async_copy(
                        bufs[1], acc.at[dst_v.at[r - 1]], sem_s1).wait()
                pltpu.async_copy(x_hbm.at[src_v.at[r + 1]], bufs[1], sem_g1)

                pltpu.make_async_copy(
                    x_hbm.at[src_v.at[r]], bufs[0], sem_g0).wait()
                _scale_chunk(bufs[0], r)
                pltpu.async_copy(bufs[0], acc.at[dst_v.at[r]], sem_s0,
                                 add=True)

                pltpu.make_async_copy(
                    x_hbm.at[src_v.at[r + 1]], bufs[1], sem_g1).wait()
                _scale_chunk(bufs[1], r + 1)
                pltpu.async_copy(bufs[1], acc.at[dst_v.at[r + 1]], sem_s1,
                                 add=True)

        for r in range(BLKCH - 2, BLKCH):
            pltpu.make_async_copy(
                bufs[r % 2], acc.at[dst_v.at[r]], ssems[r % 2]).wait()

        plsc.subcore_barrier()

        for i in range(rows_per_tile // CH):
            sl = pl.ds(base + i * CH, CH)
            pltpu.sync_copy(acc.at[sl], out_hbm.at[c].at[sl])
        pltpu.sync_copy(ws_v, ws_hbm.at[wid])

    return k(x, src3, dst3, w3, zrows)


def _combine_body(x_ref, p_ref, ws_ref, wst_ref, wnt_ref, bs_ref, bn_ref, o_ref):
    neigh = p_ref[0] + p_ref[1]
    wsum = jnp.sum(ws_ref[...], axis=0)[:, None]
    h = neigh / jnp.maximum(wsum, 1.0)
    o_ref[...] = (
        jnp.dot(x_ref[...], wst_ref[...], preferred_element_type=jnp.float32)
        + jnp.dot(h, wnt_ref[...], preferred_element_type=jnp.float32)
        + bs_ref[...] + bn_ref[...]
    ) * 0.5


def _combine(npad, xp, partials, wspart, WsT, WnT, bs, bn):
    blk = 1024
    return pl.pallas_call(
        _combine_body,
        grid=(npad // blk,),
        in_specs=[
            pl.BlockSpec((blk, D), lambda i: (i, 0)),
            pl.BlockSpec((2, blk, D), lambda i: (0, i, 0)),
            pl.BlockSpec((NW, blk), lambda i: (0, i)),
            pl.BlockSpec((D, D), lambda i: (0, 0)),
            pl.BlockSpec((D, D), lambda i: (0, 0)),
            pl.BlockSpec((1, D), lambda i: (0, 0)),
            pl.BlockSpec((1, D), lambda i: (0, 0)),
        ],
        out_specs=pl.BlockSpec((blk, D), lambda i: (i, 0)),
        out_shape=jax.ShapeDtypeStruct((npad, D), jnp.float32),
    )(xp, partials, wspart, WsT, WnT, bs, bn)


def kernel(x, edge_index, edge_weight, W_self, b_self, W_neigh, b_neigh):
    n = x.shape[0]
    e = edge_weight.shape[0]

    per_tile = -(-e // (NW * CH * BLKCH)) * CH * BLKCH
    nchunk = per_tile // CH
    epad = NW * per_tile
    npad = -(-n // (16 * CH)) * (16 * CH)

    fill = jnp.arange(epad - e, dtype=jnp.int32) % n
    src3 = jnp.concatenate([edge_index[0], fill]).reshape(NW, nchunk, CH)
    dst3 = jnp.concatenate([edge_index[1], fill]).reshape(NW, nchunk, CH)
    w3 = jnp.pad(edge_weight, (0, epad - e)).reshape(NW, nchunk, CH)
    zrows = jnp.zeros((CH, D), jnp.float32)

    partials, wspart = _sc_segsum(nchunk, npad, x, src3, dst3, w3, zrows)

    xp = jnp.pad(x, ((0, npad - n), (0, 0)))
    out = _combine(npad, xp, partials, wspart, W_self.T, W_neigh.T,
                   b_self[None, :], b_neigh[None, :])
    return out[:n]


# speedup vs baseline: 8.9503x; 1.0381x over previous
import dataclasses
import functools

import jax
import jax.numpy as jnp
from jax import lax
from jax.experimental import pallas as pl
from jax.experimental.pallas import tpu as pltpu
from jax.experimental.pallas import tpu_sc as plsc

D = 128
NW = 32
CH = 128
BLKCH = 8
LANES = 16

_BCAST_DNUMS = lax.GatherDimensionNumbers(
    offset_dims=(), collapsed_slice_dims=(0,), start_index_map=(0,))


def _bcast_lane(v16, k):
    idx = jnp.full((LANES, 1), k, jnp.int32)
    return lax.gather(v16, idx, _BCAST_DNUMS, slice_sizes=(1,),
                      mode=lax.GatherScatterMode.PROMISE_IN_BOUNDS)


def _sc_segsum(nchunk, npad, wsn, x, src3, dst3, w3, zrows):
    mesh = plsc.VectorSubcoreMesh(core_axis_name="c", subcore_axis_name="s")
    rows_per_tile = npad // 16
    nblk2 = nchunk // BLKCH // 2
    cp = pltpu.CompilerParams()
    if "needs_layout_passes" in pltpu.CompilerParams.__dataclass_fields__:
        cp = dataclasses.replace(cp, needs_layout_passes=False)

    @functools.partial(
        pl.kernel,
        out_type=(
            jax.ShapeDtypeStruct((2, npad, D), jnp.float32),
            jax.ShapeDtypeStruct((NW, npad), jnp.float32),
        ),
        mesh=mesh,
        scratch_types=[
            pltpu.VMEM((2, BLKCH, CH), jnp.int32),
            pltpu.VMEM((2, BLKCH, CH), jnp.int32),
            pltpu.VMEM((2, BLKCH, CH), jnp.float32),
            pltpu.VMEM((CH, D), jnp.float32),
            pltpu.VMEM((CH, D), jnp.float32),
            pltpu.VMEM((wsn,), jnp.float32),
            pltpu.VMEM_SHARED((npad, D), jnp.float32),
            pltpu.SemaphoreType.DMA,
            pltpu.SemaphoreType.DMA,
            pltpu.SemaphoreType.DMA,
            pltpu.SemaphoreType.DMA,
            pltpu.SemaphoreType.DMA,
        ],
        compiler_params=cp,
    )
    def k(x_hbm, src_hbm, dst_hbm, w_hbm, z_hbm, out_hbm, ws_hbm,
          src_v, dst_v, w_v, rows0_v, rows1_v, ws_v, acc,
          sem_g0, sem_g1, sem_s0, sem_s1, sem_ix):
        c = lax.axis_index("c")
        s = lax.axis_index("s")
        wid = c * 16 + s

        base = s * rows_per_tile
        for i in range(rows_per_tile // CH):
            pltpu.sync_copy(z_hbm, acc.at[pl.ds(base + i * CH, CH)])

        zv = jnp.zeros((LANES,), jnp.float32)

        @pl.loop(0, wsn, step=LANES)
        def _(i):
            ws_v[pl.ds(i, LANES)] = zv

        pltpu.sync_copy(src_hbm.at[wid].at[pl.ds(0, BLKCH)], src_v.at[0])
        pltpu.sync_copy(dst_hbm.at[wid].at[pl.ds(0, BLKCH)], dst_v.at[0])
        pltpu.sync_copy(w_hbm.at[wid].at[pl.ds(0, BLKCH)], w_v.at[0])

        plsc.subcore_barrier()

        lane0 = lax.iota(jnp.int32, LANES) == 0

        bufs = (rows0_v, rows1_v)
        gsems = (sem_g0, sem_g1)
        ssems = (sem_s0, sem_s1)

        def _scale_chunk(buf, sv, wv_blk, dv_blk, r):
            @pl.loop(0, CH, step=LANES)
            def _(g):
                w16 = wv_blk[r, pl.ds(g, LANES)]
                d16 = dv_blk[r, pl.ds(g, LANES)]
                for k in range(LANES):
                    wv = _bcast_lane(w16, k)
                    dv = _bcast_lane(d16, k)
                    e = g + k
                    for t in range(D // LANES):
                        sl = pl.ds(t * LANES, LANES)
                        buf[e, sl] = buf[e, sl] * wv
                    plsc.addupdate_scatter(ws_v, [dv], wv, mask=lane0)

        @pl.loop(0, nblk2)
        def _(p):
            for phase in (0, 1):
                b = 2 * p + phase
                sv = src_v.at[phase]
                dv_blk = dst_v.at[phase]
                wv_blk = w_v.at[phase]

                first = (p == 0) if phase == 0 else False
                if phase == 1:
                    for r in range(BLKCH - 2, BLKCH):
                        pltpu.make_async_copy(
                            bufs[r % 2], acc.at[dv_blk.at[r]],
                            ssems[r % 2]).wait()
                else:
                    @pl.when(p > 0)
                    def _():
                        for r in range(BLKCH - 2, BLKCH):
                            pltpu.make_async_copy(
                                bufs[r % 2], acc.at[dv_blk.at[r]],
                                ssems[r % 2]).wait()

                nxt = pl.ds((b + 1) * BLKCH, BLKCH)

                def _prefetch():
                    pltpu.async_copy(src_hbm.at[wid].at[nxt],
                                     src_v.at[1 - phase], sem_ix)
                    pltpu.async_copy(dst_hbm.at[wid].at[nxt],
                                     dst_v.at[1 - phase], sem_ix)
                    pltpu.async_copy(w_hbm.at[wid].at[nxt],
                                     w_v.at[1 - phase], sem_ix)

                def _prefetch_wait():
                    for _ in range(3):
                        pltpu.make_async_copy(
                            src_hbm.at[wid].at[nxt], src_v.at[1 - phase],
                            sem_ix).wait()

                if phase == 0:
                    @pl.when(p > 0)
                    def _():
                        _prefetch_wait()
                    _prefetch()
                else:
                    _prefetch_wait()
                    @pl.when(p < nblk2 - 1)
                    def _():
                        _prefetch()

                for r in range(0, BLKCH, 2):
                    if r >= 2:
                        pltpu.make_async_copy(
                            bufs[0], acc.at[dv_blk.at[r - 2]], sem_s0).wait()
                    pltpu.async_copy(x_hbm.at[sv.at[r]], bufs[0], sem_g0)
                    if r >= 2:
                        pltpu.make_async_copy(
                            bufs[1], acc.at[dv_blk.at[r - 1]], sem_s1).wait()
                    pltpu.async_copy(x_hbm.at[sv.at[r + 1]], bufs[1], sem_g1)

                    pltpu.make_async_copy(
                        x_hbm.at[sv.at[r]], bufs[0], sem_g0).wait()
                    _scale_chunk(bufs[0], sv, wv_blk, dv_blk, r)
                    pltpu.async_copy(bufs[0], acc.at[dv_blk.at[r]], sem_s0,
                                     add=True)

                    pltpu.make_async_copy(
                        x_hbm.at[sv.at[r + 1]], bufs[1], sem_g1).wait()
                    _scale_chunk(bufs[1], sv, wv_blk, dv_blk, r + 1)
                    pltpu.async_copy(bufs[1], acc.at[dv_blk.at[r + 1]],
                                     sem_s1, add=True)

        for r in range(BLKCH - 2, BLKCH):
            pltpu.make_async_copy(
                bufs[r % 2], acc.at[dst_v.at[1].at[r]], ssems[r % 2]).wait()

        plsc.subcore_barrier()

        for i in range(rows_per_tile // CH):
            sl = pl.ds(base + i * CH, CH)
            pltpu.sync_copy(acc.at[sl], out_hbm.at[c].at[sl])
        pltpu.sync_copy(ws_v, ws_hbm.at[wid].at[pl.ds(0, wsn)])

    return k(x, src3, dst3, w3, zrows)


def _self_mm_body(x_ref, wst_ref, bs_ref, o_ref):
    o_ref[...] = jnp.dot(x_ref[...], wst_ref[...],
                         preferred_element_type=jnp.float32) + bs_ref[...]


def _self_mm(npad, xp, WsT, bs):
    blk = 1024
    return pl.pallas_call(
        _self_mm_body,
        grid=(npad // blk,),
        in_specs=[
            pl.BlockSpec((blk, D), lambda i: (i, 0)),
            pl.BlockSpec((D, D), lambda i: (0, 0)),
            pl.BlockSpec((1, D), lambda i: (0, 0)),
        ],
        out_specs=pl.BlockSpec((blk, D), lambda i: (i, 0)),
        out_shape=jax.ShapeDtypeStruct((npad, D), jnp.float32),
    )(xp, WsT, bs)


def _combine_body(self_ref, p_ref, ws_ref, wnt_ref, bn_ref, o_ref):
    neigh = p_ref[0] + p_ref[1]
    wsum = jnp.sum(ws_ref[...], axis=0)[:, None]
    h = neigh / jnp.maximum(wsum, 1.0)
    o_ref[...] = (
        self_ref[...]
        + jnp.dot(h, wnt_ref[...], preferred_element_type=jnp.float32)
        + bn_ref[...]
    ) * 0.5


def _combine(npad, selfp, partials, wspart, WnT, bn):
    blk = 1024
    return pl.pallas_call(
        _combine_body,
        grid=(npad // blk,),
        in_specs=[
            pl.BlockSpec((blk, D), lambda i: (i, 0)),
            pl.BlockSpec((2, blk, D), lambda i: (0, i, 0)),
            pl.BlockSpec((NW, blk), lambda i: (0, i)),
            pl.BlockSpec((D, D), lambda i: (0, 0)),
            pl.BlockSpec((1, D), lambda i: (0, 0)),
        ],
        out_specs=pl.BlockSpec((blk, D), lambda i: (i, 0)),
        out_shape=jax.ShapeDtypeStruct((npad, D), jnp.float32),
    )(selfp, partials, wspart, WnT, bn)


def kernel(x, edge_index, edge_weight, W_self, b_self, W_neigh, b_neigh):
    n = x.shape[0]
    e = edge_weight.shape[0]

    per_tile = -(-e // (NW * CH * BLKCH * 2)) * CH * BLKCH * 2
    nchunk = per_tile // CH
    epad = NW * per_tile
    npad = -(-n // (16 * CH)) * (16 * CH)
    wsn = -(-n // CH) * CH

    fill = jnp.arange(epad - e, dtype=jnp.int32) % n
    src3 = jnp.concatenate([edge_index[0], fill]).reshape(NW, nchunk, CH)
    dst3 = jnp.concatenate([edge_index[1], fill]).reshape(NW, nchunk, CH)
    w3 = jnp.pad(edge_weight, (0, epad - e)).reshape(NW, nchunk, CH)
    zrows = jnp.zeros((CH, D), jnp.float32)

    partials, wspart = _sc_segsum(nchunk, npad, wsn, x, src3, dst3, w3, zrows)

    xp = jnp.pad(x, ((0, npad - n), (0, 0)))
    selfp = _self_mm(npad, xp, W_self.T, b_self[None, :])
    out = _combine(npad, selfp, partials, wspart, W_neigh.T, b_neigh[None, :])
    return out[:n]
